# baseline (speedup 1.0000x reference)
import functools

import jax
import jax.numpy as jnp
from jax import lax
from jax.experimental import pallas as pl
from jax.experimental.pallas import tpu as pltpu
from jax.experimental.pallas import tpu_sc as plsc

_N = 10000
_NP = 10240
_K = 16
_R = 4
_NBIG = 2 ** 30


def _knn_body(n_valid, k, br, ct, x_ref, f_ref, out_ref, dist_ref, vf_ref, tf_ref):
    T = f_ref.shape[0] // ct
    i = pl.program_id(0)
    x = x_ref[...]
    sqx = jnp.sum(x * x, axis=1, keepdims=True)
    row_ids = i * br + lax.broadcasted_iota(jnp.int32, (br, ct), 0)
    ones = jnp.ones((1, x.shape[1]), jnp.float32)

    def fill(t, carry):
        f = f_ref[pl.ds(t * ct, ct), :]
        p = lax.dot_general(x, f, (((1,), (1,)), ((), ())),
                            preferred_element_type=jnp.float32)
        sqf = lax.dot_general(ones, f * f, (((1,), (1,)), ((), ())),
                              precision=lax.Precision.HIGHEST,
                              preferred_element_type=jnp.float32)
        col = t * ct + lax.broadcasted_iota(jnp.int32, (br, ct), 1)
        d = sqx + sqf - 2.0 * p
        d = jnp.where((col == row_ids) | (col >= n_valid), jnp.inf, d)
        dist_ref[t] = d
        return carry

    lax.fori_loop(0, T, fill, 0)

    lane_k = lax.broadcasted_iota(jnp.int32, (br, k), 1)
    lane_ct = lax.broadcasted_iota(jnp.int32, (br, ct), 1)
    inf_plane = jnp.full((br, ct), jnp.inf, jnp.float32)
    zero_plane = jnp.zeros((br, ct), jnp.int32)
    for j in range(3):
        vf_ref[j] = inf_plane
        tf_ref[j] = zero_plane

    def fold(t, carry):
        d = dist_ref[t]
        v1, v2, v3 = vf_ref[0], vf_ref[1], vf_ref[2]
        t1, t2 = tf_ref[0], tf_ref[1]
        c1 = d < v1
        c2 = d < v2
        c3 = d < v3
        vf_ref[2] = jnp.where(c3, jnp.where(c2, v2, d), v3)
        tf_ref[2] = jnp.where(c3, jnp.where(c2, t2, t), tf_ref[2])
        vf_ref[1] = jnp.where(c2, jnp.where(c1, v1, d), v2)
        tf_ref[1] = jnp.where(c2, jnp.where(c1, t1, t), t2)
        vf_ref[0] = jnp.minimum(v1, d)
        tf_ref[0] = jnp.where(c1, t, t1)
        return carry

    lax.fori_loop(0, T, fold, 0)

    def kstep(kk, carry):
        m_prev, ik_prev, out = carry

        def scan(j, mc):
            m, ik = mc
            v = vf_ref[j]
            col = tf_ref[j] * ct + lane_ct
            valid = (v > m_prev) | ((v == m_prev) & (col > ik_prev))
            dm = jnp.where(valid, v, jnp.inf)
            tm = jnp.min(dm, axis=1, keepdims=True)
            cand = jnp.where(dm == tm, col, _NBIG)
            targ = jnp.min(cand, axis=1, keepdims=True)
            lt = tm < m
            eq = tm == m
            ik2 = jnp.where(lt, targ, jnp.where(eq, jnp.minimum(ik, targ), ik))
            return (jnp.minimum(m, tm), ik2)

        m, ik = lax.fori_loop(
            0, 3, scan,
            (jnp.full((br, 1), jnp.inf, jnp.float32),
             jnp.full((br, 1), _NBIG, jnp.int32)))
        out = jnp.where(lane_k == kk, ik, out)
        return (m, ik, out)

    m16, ik16, out = lax.fori_loop(
        0, k, kstep,
        (jnp.full((br, 1), -jnp.inf, jnp.float32),
         jnp.full((br, 1), -1, jnp.int32),
         jnp.zeros((br, k), jnp.int32)))
    out_ref[...] = out

    def ver(t, cnt):
        d = dist_ref[t]
        col = t * ct + lane_ct
        le = (d < m16) | ((d == m16) & (col <= ik16))
        return cnt + jnp.sum(le.astype(jnp.float32), axis=1, keepdims=True)

    cnt = lax.fori_loop(0, T, ver, jnp.zeros((br, 1), jnp.float32))
    bad = jnp.any(cnt != float(k))

    @pl.when(bad)
    def _fallback():
        def kstep_full(kk, carry):
            m_prev, ik_prev, out = carry

            def scan(t, mc):
                m, ik = mc
                tile = dist_ref[t]
                col = t * ct + lane_ct
                valid = (tile > m_prev) | ((tile == m_prev) & (col > ik_prev))
                dm = jnp.where(valid, tile, jnp.inf)
                tm = jnp.min(dm, axis=1, keepdims=True)
                cand = jnp.where(dm == tm, col, _NBIG)
                targ = jnp.min(cand, axis=1, keepdims=True)
                lt = tm < m
                eq = tm == m
                ik2 = jnp.where(lt, targ, jnp.where(eq, jnp.minimum(ik, targ), ik))
                return (jnp.minimum(m, tm), ik2)

            m, ik = lax.fori_loop(
                0, T, scan,
                (jnp.full((br, 1), jnp.inf, jnp.float32),
                 jnp.full((br, 1), _NBIG, jnp.int32)))
            out = jnp.where(lane_k == kk, ik, out)
            return (m, ik, out)

        _, _, out2 = lax.fori_loop(
            0, k, kstep_full,
            (jnp.full((br, 1), -jnp.inf, jnp.float32),
             jnp.full((br, 1), -1, jnp.int32),
             jnp.zeros((br, k), jnp.int32)))
        out_ref[...] = out2


def _knn_pallas(feat_p, n_valid, k=_K, br=128, ct=512):
    npad, lanes = feat_p.shape
    T = npad // ct
    body = functools.partial(_knn_body, n_valid, k, br, ct)
    return pl.pallas_call(
        body,
        grid=(npad // br,),
        in_specs=[pl.BlockSpec((br, lanes), lambda i: (i, 0)),
                  pl.BlockSpec((npad, lanes), lambda i: (0, 0))],
        out_specs=pl.BlockSpec((br, k), lambda i: (i, 0)),
        out_shape=jax.ShapeDtypeStruct((npad, k), jnp.int32),
        scratch_shapes=[pltpu.VMEM((T, br, ct), jnp.float32),
                        pltpu.VMEM((3, br, ct), jnp.float32),
                        pltpu.VMEM((3, br, ct), jnp.int32)],
    )(feat_p, feat_p)


def _lina_body(f_ref, wt_ref, b_ref, a_ref):
    a_ref[...] = lax.dot_general(f_ref[...], wt_ref[...], (((1,), (0,)), ((), ())),
                                 preferred_element_type=jnp.float32) + b_ref[...]


def _lina_pallas(feat_p, wt, b, br=1024):
    npad, lanes = feat_p.shape
    cp = wt.shape[1]
    return pl.pallas_call(
        _lina_body,
        grid=(npad // br,),
        in_specs=[pl.BlockSpec((br, lanes), lambda i: (i, 0)),
                  pl.BlockSpec((lanes, cp), lambda i: (0, 0)),
                  pl.BlockSpec((1, cp), lambda i: (0, 0))],
        out_specs=pl.BlockSpec((br, cp), lambda i: (i, 0)),
        out_shape=jax.ShapeDtypeStruct((npad, cp), jnp.float32),
    )(feat_p, wt, b.reshape(1, cp))


def _gather_diff_sc(feat_p, idx2d, cl, gn=8):
    npad, lanes = feat_p.shape
    info = plsc.get_sparse_core_info()
    nw = info.num_cores * info.num_subcores
    per_w = npad // nw
    g_count = per_w // gn
    mesh = plsc.VectorSubcoreMesh(core_axis_name="c", subcore_axis_name="s")

    @functools.partial(
        pl.kernel, mesh=mesh,
        out_type=jax.ShapeDtypeStruct((_K, npad, cl), jnp.float32),
        scratch_types=[
            pltpu.VMEM((g_count, gn * _K), jnp.int32),
            pltpu.VMEM((gn, lanes), jnp.float32),
            pltpu.VMEM((gn * _K, lanes), jnp.float32),
            pltpu.VMEM((_K, gn, cl), jnp.float32),
            pltpu.SemaphoreType.DMA,
        ],
    )
    def sc_kernel(f_hbm, idx_hbm, d_hbm, idx_v, xi_v, rows_v, o_v, sem):
        c = lax.axis_index("c")
        s = lax.axis_index("s")
        wid = s * info.num_cores + c
        base = wid * per_w
        pltpu.sync_copy(idx_hbm.at[pl.ds(wid * g_count, g_count)], idx_v)

        def gstep(g, carry):
            pltpu.async_copy(f_hbm.at[idx_v.at[g]], rows_v, sem).wait()
            pltpu.sync_copy(f_hbm.at[pl.ds(base + g * gn, gn)], xi_v)
            for n in range(gn):
                for cc in range(cl // 16):
                    sl = pl.ds(cc * 16, 16)
                    xi = xi_v[n, sl]
                    for kk in range(_K):
                        o_v[kk, n, sl] = rows_v[n * _K + kk, sl] - xi
            for kk in range(_K):
                pltpu.sync_copy(o_v.at[kk], d_hbm.at[kk, pl.ds(base + g * gn, gn)])
            return carry

        lax.fori_loop(0, g_count, gstep, 0)

    return sc_kernel(feat_p, idx2d)


def _edge_body(nk, a_ref, d_ref, wb_ref, o_ref):
    wb = wb_ref[...]
    m = lax.dot_general(d_ref[0], wb, (((1,), (0,)), ((), ())),
                        preferred_element_type=jnp.float32)

    def kstep(kk, m):
        e = lax.dot_general(d_ref[kk], wb, (((1,), (0,)), ((), ())),
                            preferred_element_type=jnp.float32)
        return jnp.maximum(m, e)

    m = lax.fori_loop(1, nk, kstep, m)
    o_ref[...] = jnp.maximum(a_ref[...] + m, 0.0)


def _edge_pallas(a_arr, d_arr, wb, br=512):
    npad, cp = a_arr.shape
    nk, _, cl = d_arr.shape
    body = functools.partial(_edge_body, nk)
    return pl.pallas_call(
        body,
        grid=(npad // br,),
        in_specs=[pl.BlockSpec((br, cp), lambda i: (i, 0)),
                  pl.BlockSpec((nk, br, cl), lambda i: (0, i, 0)),
                  pl.BlockSpec((cl, cp), lambda i: (0, 0))],
        out_specs=pl.BlockSpec((br, cp), lambda i: (i, 0)),
        out_shape=jax.ShapeDtypeStruct((npad, cp), jnp.float32),
    )(a_arr, d_arr, wb)


def _mlp_body(h_ref, w1_ref, b1_ref, w2_ref, b2_ref, o_ref):
    t = lax.dot_general(h_ref[...], w1_ref[...], (((1,), (0,)), ((), ())),
                        preferred_element_type=jnp.float32) + b1_ref[...]
    t = jnp.maximum(t, 0.0)
    o_ref[...] = lax.dot_general(t, w2_ref[...], (((1,), (0,)), ((), ())),
                                 preferred_element_type=jnp.float32) + b2_ref[...]


def _mlp_pallas(h, w1, b1, w2, b2, br=1024):
    m, cin = h.shape
    ch = w1.shape[1]
    cout = w2.shape[1]
    return pl.pallas_call(
        _mlp_body,
        grid=(m // br,),
        in_specs=[pl.BlockSpec((br, cin), lambda i: (i, 0)),
                  pl.BlockSpec((cin, ch), lambda i: (0, 0)),
                  pl.BlockSpec((1, ch), lambda i: (0, 0)),
                  pl.BlockSpec((ch, cout), lambda i: (0, 0)),
                  pl.BlockSpec((1, cout), lambda i: (0, 0))],
        out_specs=pl.BlockSpec((br, cout), lambda i: (i, 0)),
        out_shape=jax.ShapeDtypeStruct((m, cout), jnp.float32),
    )(h, w1, b1.reshape(1, ch), w2, b2.reshape(1, cout))


def _dynconv(feat128, w, b, cin):
    cout = w.shape[1]
    cp = max(cout, 128)
    cl = max(cin, 16)
    wt, wb = w[:cin], w[cin:]
    wtp = jnp.pad(wt, ((0, 128 - cin), (0, cp - cout)))
    wbp = jnp.pad(wb, ((0, cl - cin), (0, cp - cout)))
    bp = jnp.pad(b, (0, cp - cout))
    idx = _knn_pallas(feat128, _N)
    a_arr = _lina_pallas(feat128, wtp, bp)
    idx2d = idx.reshape(_NP * _K // 128, 128)
    d_arr = _gather_diff_sc(feat128, idx2d, cl)
    return _edge_pallas(a_arr, d_arr, wbp)


def kernel(x, W0, b0, W1, b1, Wns, bns, Wr1, br1, Wr2, br2):
    f0 = jnp.pad(x, ((0, _NP - _N), (0, 128 - x.shape[1])))
    h0 = _dynconv(f0, W0, b0, 3)
    h1 = _dynconv(h0, W1, b1, 32)
    h2 = _dynconv(h1, Wns, bns, 64)
    hr = h2.reshape(_NP * _R, 64)
    out = _mlp_pallas(hr, Wr1, br1, Wr2, br2)
    return out[: _N * _R]


# speedup vs baseline: 1.5281x; 1.5281x over previous
import functools

import jax
import jax.numpy as jnp
from jax import lax
from jax.experimental import pallas as pl
from jax.experimental.pallas import tpu as pltpu
from jax.experimental.pallas import tpu_sc as plsc

_N = 10000
_NP = 10240
_K = 16
_R = 4
_NBIG = 2 ** 30


def _knn_body(n_valid, k, br, ct, x_ref, f_ref, out_ref, dist_ref, vf_ref, tf_ref):
    T = f_ref.shape[0] // ct
    i = pl.program_id(0)
    x = x_ref[...]
    sqx = jnp.sum(x * x, axis=1, keepdims=True)
    row_ids = i * br + lax.broadcasted_iota(jnp.int32, (br, ct), 0)
    ones = jnp.ones((1, x.shape[1]), jnp.float32)

    def fill(t, carry):
        f = f_ref[pl.ds(t * ct, ct), :]
        p = lax.dot_general(x, f, (((1,), (1,)), ((), ())),
                            preferred_element_type=jnp.float32)
        sqf = lax.dot_general(ones, f * f, (((1,), (1,)), ((), ())),
                              precision=lax.Precision.HIGHEST,
                              preferred_element_type=jnp.float32)
        col = t * ct + lax.broadcasted_iota(jnp.int32, (br, ct), 1)
        d = sqx + sqf - 2.0 * p
        d = jnp.where((col == row_ids) | (col >= n_valid), jnp.inf, d)
        dist_ref[t] = d
        return carry

    lax.fori_loop(0, T, fill, 0)

    lane_k = lax.broadcasted_iota(jnp.int32, (br, k), 1)
    lane_ct = lax.broadcasted_iota(jnp.int32, (br, ct), 1)
    inf_plane = jnp.full((br, ct), jnp.inf, jnp.float32)
    zero_plane = jnp.zeros((br, ct), jnp.int32)
    for j in range(3):
        vf_ref[j] = inf_plane
        tf_ref[j] = zero_plane

    def fold(t, carry):
        d = dist_ref[t]
        v1, v2, v3 = vf_ref[0], vf_ref[1], vf_ref[2]
        t1, t2 = tf_ref[0], tf_ref[1]
        c1 = d < v1
        c2 = d < v2
        c3 = d < v3
        vf_ref[2] = jnp.where(c3, jnp.where(c2, v2, d), v3)
        tf_ref[2] = jnp.where(c3, jnp.where(c2, t2, t), tf_ref[2])
        vf_ref[1] = jnp.where(c2, jnp.where(c1, v1, d), v2)
        tf_ref[1] = jnp.where(c2, jnp.where(c1, t1, t), t2)
        vf_ref[0] = jnp.minimum(v1, d)
        tf_ref[0] = jnp.where(c1, t, t1)
        return carry

    lax.fori_loop(0, T, fold, 0)

    def kstep(kk, carry):
        m_prev, ik_prev, out = carry

        def scan(j, mc):
            m, ik = mc
            v = vf_ref[j]
            col = tf_ref[j] * ct + lane_ct
            valid = (v > m_prev) | ((v == m_prev) & (col > ik_prev))
            dm = jnp.where(valid, v, jnp.inf)
            tm = jnp.min(dm, axis=1, keepdims=True)
            cand = jnp.where(dm == tm, col, _NBIG)
            targ = jnp.min(cand, axis=1, keepdims=True)
            lt = tm < m
            eq = tm == m
            ik2 = jnp.where(lt, targ, jnp.where(eq, jnp.minimum(ik, targ), ik))
            return (jnp.minimum(m, tm), ik2)

        m, ik = lax.fori_loop(
            0, 3, scan,
            (jnp.full((br, 1), jnp.inf, jnp.float32),
             jnp.full((br, 1), _NBIG, jnp.int32)))
        out = jnp.where(lane_k == kk, ik, out)
        return (m, ik, out)

    m16, ik16, out = lax.fori_loop(
        0, k, kstep,
        (jnp.full((br, 1), -jnp.inf, jnp.float32),
         jnp.full((br, 1), -1, jnp.int32),
         jnp.zeros((br, k), jnp.int32)))
    out_ref[...] = out

    def ver(t, cnt):
        d = dist_ref[t]
        col = t * ct + lane_ct
        le = (d < m16) | ((d == m16) & (col <= ik16))
        return cnt + jnp.sum(le.astype(jnp.float32), axis=1, keepdims=True)

    cnt = lax.fori_loop(0, T, ver, jnp.zeros((br, 1), jnp.float32))
    bad = jnp.any(cnt != float(k))

    @pl.when(bad)
    def _fallback():
        def kstep_full(kk, carry):
            m_prev, ik_prev, out = carry

            def scan(t, mc):
                m, ik = mc
                tile = dist_ref[t]
                col = t * ct + lane_ct
                valid = (tile > m_prev) | ((tile == m_prev) & (col > ik_prev))
                dm = jnp.where(valid, tile, jnp.inf)
                tm = jnp.min(dm, axis=1, keepdims=True)
                cand = jnp.where(dm == tm, col, _NBIG)
                targ = jnp.min(cand, axis=1, keepdims=True)
                lt = tm < m
                eq = tm == m
                ik2 = jnp.where(lt, targ, jnp.where(eq, jnp.minimum(ik, targ), ik))
                return (jnp.minimum(m, tm), ik2)

            m, ik = lax.fori_loop(
                0, T, scan,
                (jnp.full((br, 1), jnp.inf, jnp.float32),
                 jnp.full((br, 1), _NBIG, jnp.int32)))
            out = jnp.where(lane_k == kk, ik, out)
            return (m, ik, out)

        _, _, out2 = lax.fori_loop(
            0, k, kstep_full,
            (jnp.full((br, 1), -jnp.inf, jnp.float32),
             jnp.full((br, 1), -1, jnp.int32),
             jnp.zeros((br, k), jnp.int32)))
        out_ref[...] = out2


def _knn_pallas(feat_p, n_valid, k=_K, br=512, ct=512):
    npad, lanes = feat_p.shape
    T = npad // ct
    body = functools.partial(_knn_body, n_valid, k, br, ct)
    return pl.pallas_call(
        body,
        grid=(npad // br,),
        in_specs=[pl.BlockSpec((br, lanes), lambda i: (i, 0)),
                  pl.BlockSpec((npad, lanes), lambda i: (0, 0))],
        out_specs=pl.BlockSpec((br, k), lambda i: (i, 0)),
        out_shape=jax.ShapeDtypeStruct((npad, k), jnp.int32),
        scratch_shapes=[pltpu.VMEM((T, br, ct), jnp.float32),
                        pltpu.VMEM((3, br, ct), jnp.float32),
                        pltpu.VMEM((3, br, ct), jnp.int32)],
    )(feat_p, feat_p)


def _lina_body(f_ref, wt_ref, b_ref, a_ref):
    a_ref[...] = lax.dot_general(f_ref[...], wt_ref[...], (((1,), (0,)), ((), ())),
                                 preferred_element_type=jnp.float32) + b_ref[...]


def _lina_pallas(feat_p, wt, b, br=1024):
    npad, lanes = feat_p.shape
    cp = wt.shape[1]
    return pl.pallas_call(
        _lina_body,
        grid=(npad // br,),
        in_specs=[pl.BlockSpec((br, lanes), lambda i: (i, 0)),
                  pl.BlockSpec((lanes, cp), lambda i: (0, 0)),
                  pl.BlockSpec((1, cp), lambda i: (0, 0))],
        out_specs=pl.BlockSpec((br, cp), lambda i: (i, 0)),
        out_shape=jax.ShapeDtypeStruct((npad, cp), jnp.float32),
    )(feat_p, wt, b.reshape(1, cp))


def _gather_diff_sc(feat_p, idx2d, cl, gn=8):
    npad, lanes = feat_p.shape
    info = plsc.get_sparse_core_info()
    nw = info.num_cores * info.num_subcores
    per_w = npad // nw
    g_count = per_w // gn
    mesh = plsc.VectorSubcoreMesh(core_axis_name="c", subcore_axis_name="s")

    @functools.partial(
        pl.kernel, mesh=mesh,
        out_type=jax.ShapeDtypeStruct((_K, npad, cl), jnp.float32),
        scratch_types=[
            pltpu.VMEM((g_count, gn * _K), jnp.int32),
            pltpu.VMEM((gn, lanes), jnp.float32),
            pltpu.VMEM((gn * _K, lanes), jnp.float32),
            pltpu.VMEM((_K, gn, cl), jnp.float32),
            pltpu.SemaphoreType.DMA,
        ],
    )
    def sc_kernel(f_hbm, idx_hbm, d_hbm, idx_v, xi_v, rows_v, o_v, sem):
        c = lax.axis_index("c")
        s = lax.axis_index("s")
        wid = s * info.num_cores + c
        base = wid * per_w
        pltpu.sync_copy(idx_hbm.at[pl.ds(wid * g_count, g_count)], idx_v)

        def gstep(g, carry):
            pltpu.async_copy(f_hbm.at[idx_v.at[g]], rows_v, sem).wait()
            pltpu.sync_copy(f_hbm.at[pl.ds(base + g * gn, gn)], xi_v)
            for n in range(gn):
                for cc in range(cl // 16):
                    sl = pl.ds(cc * 16, 16)
                    xi = xi_v[n, sl]
                    for kk in range(_K):
                        o_v[kk, n, sl] = rows_v[n * _K + kk, sl] - xi
            for kk in range(_K):
                pltpu.sync_copy(o_v.at[kk], d_hbm.at[kk, pl.ds(base + g * gn, gn)])
            return carry

        lax.fori_loop(0, g_count, gstep, 0)

    return sc_kernel(feat_p, idx2d)


def _edge_body(nk, a_ref, d_ref, wb_ref, o_ref):
    wb = wb_ref[...]
    m = lax.dot_general(d_ref[0], wb, (((1,), (0,)), ((), ())),
                        preferred_element_type=jnp.float32)

    def kstep(kk, m):
        e = lax.dot_general(d_ref[kk], wb, (((1,), (0,)), ((), ())),
                            preferred_element_type=jnp.float32)
        return jnp.maximum(m, e)

    m = lax.fori_loop(1, nk, kstep, m)
    o_ref[...] = jnp.maximum(a_ref[...] + m, 0.0)


def _edge_pallas(a_arr, d_arr, wb, br=512):
    npad, cp = a_arr.shape
    nk, _, cl = d_arr.shape
    body = functools.partial(_edge_body, nk)
    return pl.pallas_call(
        body,
        grid=(npad // br,),
        in_specs=[pl.BlockSpec((br, cp), lambda i: (i, 0)),
                  pl.BlockSpec((nk, br, cl), lambda i: (0, i, 0)),
                  pl.BlockSpec((cl, cp), lambda i: (0, 0))],
        out_specs=pl.BlockSpec((br, cp), lambda i: (i, 0)),
        out_shape=jax.ShapeDtypeStruct((npad, cp), jnp.float32),
    )(a_arr, d_arr, wb)


def _mlp_body(h_ref, w1_ref, b1_ref, w2_ref, b2_ref, o_ref):
    t = lax.dot_general(h_ref[...], w1_ref[...], (((1,), (0,)), ((), ())),
                        preferred_element_type=jnp.float32) + b1_ref[...]
    t = jnp.maximum(t, 0.0)
    o_ref[...] = lax.dot_general(t, w2_ref[...], (((1,), (0,)), ((), ())),
                                 preferred_element_type=jnp.float32) + b2_ref[...]


def _mlp_pallas(h, w1, b1, w2, b2, br=1024):
    m, cin = h.shape
    ch = w1.shape[1]
    cout = w2.shape[1]
    return pl.pallas_call(
        _mlp_body,
        grid=(m // br,),
        in_specs=[pl.BlockSpec((br, cin), lambda i: (i, 0)),
                  pl.BlockSpec((cin, ch), lambda i: (0, 0)),
                  pl.BlockSpec((1, ch), lambda i: (0, 0)),
                  pl.BlockSpec((ch, cout), lambda i: (0, 0)),
                  pl.BlockSpec((1, cout), lambda i: (0, 0))],
        out_specs=pl.BlockSpec((br, cout), lambda i: (i, 0)),
        out_shape=jax.ShapeDtypeStruct((m, cout), jnp.float32),
    )(h, w1, b1.reshape(1, ch), w2, b2.reshape(1, cout))


def _dynconv(feat128, w, b, cin):
    cout = w.shape[1]
    cp = max(cout, 128)
    cl = max(cin, 16)
    wt, wb = w[:cin], w[cin:]
    wtp = jnp.pad(wt, ((0, 128 - cin), (0, cp - cout)))
    wbp = jnp.pad(wb, ((0, cl - cin), (0, cp - cout)))
    bp = jnp.pad(b, (0, cp - cout))
    idx = _knn_pallas(feat128, _N)
    a_arr = _lina_pallas(feat128, wtp, bp)
    idx2d = idx.reshape(_NP * _K // 128, 128)
    d_arr = _gather_diff_sc(feat128, idx2d, cl)
    return _edge_pallas(a_arr, d_arr, wbp)


def kernel(x, W0, b0, W1, b1, Wns, bns, Wr1, br1, Wr2, br2):
    f0 = jnp.pad(x, ((0, _NP - _N), (0, 128 - x.shape[1])))
    h0 = _dynconv(f0, W0, b0, 3)
    h1 = _dynconv(h0, W1, b1, 32)
    h2 = _dynconv(h1, Wns, bns, 64)
    hr = h2.reshape(_NP * _R, 64)
    out = _mlp_pallas(hr, Wr1, br1, Wr2, br2)
    return out[: _N * _R]


# speedup vs baseline: 1.5415x; 1.0087x over previous
import functools

import jax
import jax.numpy as jnp
from jax import lax
from jax.experimental import pallas as pl
from jax.experimental.pallas import tpu as pltpu
from jax.experimental.pallas import tpu_sc as plsc

_N = 10000
_NP = 10240
_K = 16
_R = 4
_NBIG = 2 ** 30


def _knn_body(n_valid, k, br, ct, x_ref, f_ref, out_ref, dist_ref, vf_ref, tf_ref):
    T = f_ref.shape[0] // ct
    i = pl.program_id(0)
    x = x_ref[...]
    sqx = jnp.sum(x * x, axis=1, keepdims=True)
    row_ids = i * br + lax.broadcasted_iota(jnp.int32, (br, ct), 0)
    ones = jnp.ones((1, x.shape[1]), jnp.float32)

    def fill(t, carry):
        f = f_ref[pl.ds(t * ct, ct), :]
        p = lax.dot_general(x, f, (((1,), (1,)), ((), ())),
                            preferred_element_type=jnp.float32)
        sqf = lax.dot_general(ones, f * f, (((1,), (1,)), ((), ())),
                              precision=lax.Precision.HIGHEST,
                              preferred_element_type=jnp.float32)
        col = t * ct + lax.broadcasted_iota(jnp.int32, (br, ct), 1)
        d = sqx + sqf - 2.0 * p
        d = jnp.where((col == row_ids) | (col >= n_valid), jnp.inf, d)
        dist_ref[t] = d
        return carry

    lax.fori_loop(0, T, fill, 0)

    lane_k = lax.broadcasted_iota(jnp.int32, (br, k), 1)
    lane_ct = lax.broadcasted_iota(jnp.int32, (br, ct), 1)
    inf_plane = jnp.full((br, ct), jnp.inf, jnp.float32)
    zero_plane = jnp.zeros((br, ct), jnp.int32)
    for j in range(3):
        vf_ref[j] = inf_plane
        tf_ref[j] = zero_plane

    def fold(t, carry):
        d = dist_ref[t]
        v1, v2, v3 = vf_ref[0], vf_ref[1], vf_ref[2]
        t1, t2 = tf_ref[0], tf_ref[1]
        c1 = d < v1
        c2 = d < v2
        c3 = d < v3
        vf_ref[2] = jnp.where(c3, jnp.where(c2, v2, d), v3)
        tf_ref[2] = jnp.where(c3, jnp.where(c2, t2, t), tf_ref[2])
        vf_ref[1] = jnp.where(c2, jnp.where(c1, v1, d), v2)
        tf_ref[1] = jnp.where(c2, jnp.where(c1, t1, t), t2)
        vf_ref[0] = jnp.minimum(v1, d)
        tf_ref[0] = jnp.where(c1, t, t1)
        return carry

    lax.fori_loop(0, T, fold, 0)

    def kstep(kk, carry):
        m_prev, ik_prev, out = carry

        def scan(j, mc):
            m, ik = mc
            v = vf_ref[j]
            col = tf_ref[j] * ct + lane_ct
            valid = (v > m_prev) | ((v == m_prev) & (col > ik_prev))
            dm = jnp.where(valid, v, jnp.inf)
            tm = jnp.min(dm, axis=1, keepdims=True)
            cand = jnp.where(dm == tm, col, _NBIG)
            targ = jnp.min(cand, axis=1, keepdims=True)
            lt = tm < m
            eq = tm == m
            ik2 = jnp.where(lt, targ, jnp.where(eq, jnp.minimum(ik, targ), ik))
            return (jnp.minimum(m, tm), ik2)

        m, ik = lax.fori_loop(
            0, 3, scan,
            (jnp.full((br, 1), jnp.inf, jnp.float32),
             jnp.full((br, 1), _NBIG, jnp.int32)))
        out = jnp.where(lane_k == kk, ik, out)
        return (m, ik, out)

    m16, ik16, out = lax.fori_loop(
        0, k, kstep,
        (jnp.full((br, 1), -jnp.inf, jnp.float32),
         jnp.full((br, 1), -1, jnp.int32),
         jnp.zeros((br, k), jnp.int32)))
    out_ref[...] = out

    def ver(t, cnt):
        d = dist_ref[t]
        col = t * ct + lane_ct
        le = (d < m16) | ((d == m16) & (col <= ik16))
        return cnt + jnp.sum(le.astype(jnp.float32), axis=1, keepdims=True)

    cnt = lax.fori_loop(0, T, ver, jnp.zeros((br, 1), jnp.float32))
    bad = jnp.any(cnt != float(k))

    @pl.when(bad)
    def _fallback():
        def kstep_full(kk, carry):
            m_prev, ik_prev, out = carry

            def scan(t, mc):
                m, ik = mc
                tile = dist_ref[t]
                col = t * ct + lane_ct
                valid = (tile > m_prev) | ((tile == m_prev) & (col > ik_prev))
                dm = jnp.where(valid, tile, jnp.inf)
                tm = jnp.min(dm, axis=1, keepdims=True)
                cand = jnp.where(dm == tm, col, _NBIG)
                targ = jnp.min(cand, axis=1, keepdims=True)
                lt = tm < m
                eq = tm == m
                ik2 = jnp.where(lt, targ, jnp.where(eq, jnp.minimum(ik, targ), ik))
                return (jnp.minimum(m, tm), ik2)

            m, ik = lax.fori_loop(
                0, T, scan,
                (jnp.full((br, 1), jnp.inf, jnp.float32),
                 jnp.full((br, 1), _NBIG, jnp.int32)))
            out = jnp.where(lane_k == kk, ik, out)
            return (m, ik, out)

        _, _, out2 = lax.fori_loop(
            0, k, kstep_full,
            (jnp.full((br, 1), -jnp.inf, jnp.float32),
             jnp.full((br, 1), -1, jnp.int32),
             jnp.zeros((br, k), jnp.int32)))
        out_ref[...] = out2


def _knn_pallas(feat_p, n_valid, k=_K, br=640, ct=512):
    npad, lanes = feat_p.shape
    T = npad // ct
    body = functools.partial(_knn_body, n_valid, k, br, ct)
    return pl.pallas_call(
        body,
        grid=(npad // br,),
        in_specs=[pl.BlockSpec((br, lanes), lambda i: (i, 0)),
                  pl.BlockSpec((npad, lanes), lambda i: (0, 0))],
        out_specs=pl.BlockSpec((br, k), lambda i: (i, 0)),
        out_shape=jax.ShapeDtypeStruct((npad, k), jnp.int32),
        scratch_shapes=[pltpu.VMEM((T, br, ct), jnp.float32),
                        pltpu.VMEM((3, br, ct), jnp.float32),
                        pltpu.VMEM((3, br, ct), jnp.int32)],
    )(feat_p, feat_p)


def _lina_body(f_ref, wt_ref, b_ref, a_ref):
    a_ref[...] = lax.dot_general(f_ref[...], wt_ref[...], (((1,), (0,)), ((), ())),
                                 preferred_element_type=jnp.float32) + b_ref[...]


def _lina_pallas(feat_p, wt, b, br=1024):
    npad, lanes = feat_p.shape
    cp = wt.shape[1]
    return pl.pallas_call(
        _lina_body,
        grid=(npad // br,),
        in_specs=[pl.BlockSpec((br, lanes), lambda i: (i, 0)),
                  pl.BlockSpec((lanes, cp), lambda i: (0, 0)),
                  pl.BlockSpec((1, cp), lambda i: (0, 0))],
        out_specs=pl.BlockSpec((br, cp), lambda i: (i, 0)),
        out_shape=jax.ShapeDtypeStruct((npad, cp), jnp.float32),
    )(feat_p, wt, b.reshape(1, cp))


def _gather_diff_sc(feat_p, idx2d, cl, gn=8):
    npad, lanes = feat_p.shape
    info = plsc.get_sparse_core_info()
    nw = info.num_cores * info.num_subcores
    per_w = npad // nw
    g_count = per_w // gn
    mesh = plsc.VectorSubcoreMesh(core_axis_name="c", subcore_axis_name="s")

    @functools.partial(
        pl.kernel, mesh=mesh,
        out_type=jax.ShapeDtypeStruct((_K, npad, cl), jnp.float32),
        scratch_types=[
            pltpu.VMEM((g_count, gn * _K), jnp.int32),
            pltpu.VMEM((gn, lanes), jnp.float32),
            pltpu.VMEM((gn * _K, lanes), jnp.float32),
            pltpu.VMEM((_K, gn, cl), jnp.float32),
            pltpu.SemaphoreType.DMA,
        ],
    )
    def sc_kernel(f_hbm, idx_hbm, d_hbm, idx_v, xi_v, rows_v, o_v, sem):
        c = lax.axis_index("c")
        s = lax.axis_index("s")
        wid = s * info.num_cores + c
        base = wid * per_w
        pltpu.sync_copy(idx_hbm.at[pl.ds(wid * g_count, g_count)], idx_v)

        def gstep(g, carry):
            pltpu.async_copy(f_hbm.at[idx_v.at[g]], rows_v, sem).wait()
            pltpu.sync_copy(f_hbm.at[pl.ds(base + g * gn, gn)], xi_v)
            for n in range(gn):
                for cc in range(cl // 16):
                    sl = pl.ds(cc * 16, 16)
                    xi = xi_v[n, sl]
                    for kk in range(_K):
                        o_v[kk, n, sl] = rows_v[n * _K + kk, sl] - xi
            for kk in range(_K):
                pltpu.sync_copy(o_v.at[kk], d_hbm.at[kk, pl.ds(base + g * gn, gn)])
            return carry

        lax.fori_loop(0, g_count, gstep, 0)

    return sc_kernel(feat_p, idx2d)


def _edge_body(nk, a_ref, d_ref, wb_ref, o_ref):
    wb = wb_ref[...]
    m = lax.dot_general(d_ref[0], wb, (((1,), (0,)), ((), ())),
                        preferred_element_type=jnp.float32)

    def kstep(kk, m):
        e = lax.dot_general(d_ref[kk], wb, (((1,), (0,)), ((), ())),
                            preferred_element_type=jnp.float32)
        return jnp.maximum(m, e)

    m = lax.fori_loop(1, nk, kstep, m)
    o_ref[...] = jnp.maximum(a_ref[...] + m, 0.0)


def _edge_pallas(a_arr, d_arr, wb, br=512):
    npad, cp = a_arr.shape
    nk, _, cl = d_arr.shape
    body = functools.partial(_edge_body, nk)
    return pl.pallas_call(
        body,
        grid=(npad // br,),
        in_specs=[pl.BlockSpec((br, cp), lambda i: (i, 0)),
                  pl.BlockSpec((nk, br, cl), lambda i: (0, i, 0)),
                  pl.BlockSpec((cl, cp), lambda i: (0, 0))],
        out_specs=pl.BlockSpec((br, cp), lambda i: (i, 0)),
        out_shape=jax.ShapeDtypeStruct((npad, cp), jnp.float32),
    )(a_arr, d_arr, wb)


def _mlp_body(h_ref, w1_ref, b1_ref, w2_ref, b2_ref, o_ref):
    t = lax.dot_general(h_ref[...], w1_ref[...], (((1,), (0,)), ((), ())),
                        preferred_element_type=jnp.float32) + b1_ref[...]
    t = jnp.maximum(t, 0.0)
    o_ref[...] = lax.dot_general(t, w2_ref[...], (((1,), (0,)), ((), ())),
                                 preferred_element_type=jnp.float32) + b2_ref[...]


def _mlp_pallas(h, w1, b1, w2, b2, br=1024):
    m, cin = h.shape
    ch = w1.shape[1]
    cout = w2.shape[1]
    return pl.pallas_call(
        _mlp_body,
        grid=(m // br,),
        in_specs=[pl.BlockSpec((br, cin), lambda i: (i, 0)),
                  pl.BlockSpec((cin, ch), lambda i: (0, 0)),
                  pl.BlockSpec((1, ch), lambda i: (0, 0)),
                  pl.BlockSpec((ch, cout), lambda i: (0, 0)),
                  pl.BlockSpec((1, cout), lambda i: (0, 0))],
        out_specs=pl.BlockSpec((br, cout), lambda i: (i, 0)),
        out_shape=jax.ShapeDtypeStruct((m, cout), jnp.float32),
    )(h, w1, b1.reshape(1, ch), w2, b2.reshape(1, cout))


def _dynconv(feat128, w, b, cin):
    cout = w.shape[1]
    cp = max(cout, 128)
    cl = max(cin, 16)
    wt, wb = w[:cin], w[cin:]
    wtp = jnp.pad(wt, ((0, 128 - cin), (0, cp - cout)))
    wbp = jnp.pad(wb, ((0, cl - cin), (0, cp - cout)))
    bp = jnp.pad(b, (0, cp - cout))
    idx = _knn_pallas(feat128, _N)
    a_arr = _lina_pallas(feat128, wtp, bp)
    idx2d = idx.reshape(_NP * _K // 128, 128)
    d_arr = _gather_diff_sc(feat128, idx2d, cl)
    return _edge_pallas(a_arr, d_arr, wbp)


def kernel(x, W0, b0, W1, b1, Wns, bns, Wr1, br1, Wr2, br2):
    f0 = jnp.pad(x, ((0, _NP - _N), (0, 128 - x.shape[1])))
    h0 = _dynconv(f0, W0, b0, 3)
    h1 = _dynconv(h0, W1, b1, 32)
    h2 = _dynconv(h1, Wns, bns, 64)
    hr = h2.reshape(_NP * _R, 64)
    out = _mlp_pallas(hr, Wr1, br1, Wr2, br2)
    return out[: _N * _R]
